# Initial kernel scaffold; baseline (speedup 1.0000x reference)
import jax
import jax.numpy as jnp
from jax.experimental import pallas as pl


def kernel(adj_matrix, node_feats, count_matrix, library_size, slice_label, basis, params):
    raise NotImplementedError("write your pallas kernel here")


# speedup vs baseline: 1.9705x; 1.9705x over previous
import jax
import jax.numpy as jnp
from jax.experimental import pallas as pl
from jax.experimental.pallas import tpu as pltpu

_N = 10000
_BM = 1000
_NI = _N // _BM
_BK = 1024
_NK = -(-_N // _BK)
_NP = _NK * _BK
_IN = 512
_HID = 512
_LAT = 128
_NCT = 20
_NSL = 8
_SLE = 16
_COEF_FE = 0.1


def _proj_body(x_ref, W_ref, b_ref, v0_ref, v1_ref, h_ref, f1_ref, f2t_ref):
    i = pl.program_id(0)
    h = jax.lax.dot_general(x_ref[...], W_ref[...], (((1,), (1,)), ((), ())),
                            preferred_element_type=jnp.float32) + b_ref[...]
    rows = i * _BK + jax.lax.broadcasted_iota(jnp.int32, (_BK, 1), 0)
    h = jnp.where(rows < _N, h, 0.0)
    h_ref[...] = h
    f1_ref[...] = jnp.dot(h, v0_ref[...], preferred_element_type=jnp.float32)
    f2t_ref[...] = jax.lax.dot_general(v1_ref[...], h, (((0,), (1,)), ((), ())),
                                       preferred_element_type=jnp.float32)


def _attn_body(adj_ref, f1_ref, f2t_ref, h_ref, g_ref, acc_ref, den_ref):
    j = pl.program_id(1)

    @pl.when(j == 0)
    def _init():
        acc_ref[...] = jnp.zeros_like(acc_ref)
        den_ref[...] = jnp.zeros_like(den_ref)

    s = f1_ref[...] + f2t_ref[...]
    vals = jax.nn.sigmoid(adj_ref[...] * s) - 0.5
    cols = j * _BK + jax.lax.broadcasted_iota(jnp.int32, (1, _BK), 1)
    keep = jnp.logical_and(vals != 0.0, cols < _N)
    e = jnp.where(keep, jnp.exp(vals), 0.0)
    hj = h_ref[pl.ds(j * _BK, _BK), :]
    acc_ref[...] += jnp.dot(e, hj, preferred_element_type=jnp.float32)
    den_ref[...] += jnp.sum(e, axis=1, keepdims=True)

    @pl.when(j == _NK - 1)
    def _fin():
        g_ref[...] = acc_ref[...] / jnp.maximum(den_ref[...], 1e-30)


def _elu(x):
    return jnp.where(x > 0, x, jnp.exp(jnp.minimum(x, 0.0)) - 1.0)


def _head_body(g_ref, x_ref, cnt_ref, lib_ref, sl_ref,
               semb_ref, gam_t_ref, basis_ref,
               e2W_ref, e2b_ref,
               d1Wz_ref, d1We_ref, d1b_ref, d2W_ref, d2b_ref,
               bWz_ref, bWe_ref, bb_ref, aWz_ref, aWe_ref, ab_ref,
               dec_ref, feat_ref):
    def dg(A, B):
        return jax.lax.dot_general(A, B, (((1,), (1,)), ((), ())),
                                   preferred_element_type=jnp.float32)

    hid = _elu(g_ref[...])
    Z = dg(hid, e2W_ref[...]) + e2b_ref[...]
    oh = (sl_ref[...] == jax.lax.broadcasted_iota(jnp.int32, (_BM, _NSL), 1)
          ).astype(jnp.float32)
    emb = jnp.dot(oh, semb_ref[...], preferred_element_type=jnp.float32)
    gam = jnp.dot(oh, gam_t_ref[...], preferred_element_type=jnp.float32)
    zZ = _elu(Z)
    zE = _elu(emb)
    beta_l = dg(zZ, bWz_ref[...]) + dg(zE, bWe_ref[...]) + bb_ref[...]
    beta_l = beta_l - jnp.max(beta_l, axis=1, keepdims=True)
    ebl = jnp.exp(beta_l)
    beta = ebl / jnp.sum(ebl, axis=1, keepdims=True)
    alpha = dg(zZ, aWz_ref[...]) + dg(zE, aWe_ref[...]) + ab_ref[...]
    d1 = _elu(dg(Z, d1Wz_ref[...]) + dg(emb, d1We_ref[...]) + d1b_ref[...])
    recon = dg(d1, d2W_ref[...]) + d2b_ref[...]
    mu = jnp.dot(beta, basis_ref[...], preferred_element_type=jnp.float32)
    lib = lib_ref[...]
    log_lam = jnp.log(lib + 1e-6) + jnp.log(mu + 1e-6) + alpha + gam
    lam = lib * mu * jnp.exp(alpha) * jnp.exp(gam)
    dec_ref[...] = jnp.sum(cnt_ref[...] * log_lam - lam).reshape(1, 1, 1)
    diff = x_ref[...] - recon
    feat_ref[...] = jnp.sum(
        jnp.sqrt(jnp.sum(diff * diff, axis=1, keepdims=True) + 1e-12)
    ).reshape(1, 1, 1)


def kernel(adj_matrix, node_feats, count_matrix, library_size, slice_label,
           basis, params):
    p = params
    f32 = jnp.float32

    h, f1, f2t = pl.pallas_call(
        _proj_body,
        grid=(_NK,),
        in_specs=[pl.BlockSpec((_BK, _IN), lambda i: (i, 0)),
                  pl.BlockSpec((_HID, _IN), lambda i: (0, 0)),
                  pl.BlockSpec((1, _HID), lambda i: (0, 0)),
                  pl.BlockSpec((_HID, 1), lambda i: (0, 0)),
                  pl.BlockSpec((_HID, 1), lambda i: (0, 0))],
        out_specs=[pl.BlockSpec((_BK, _HID), lambda i: (i, 0)),
                   pl.BlockSpec((_BK, 1), lambda i: (i, 0)),
                   pl.BlockSpec((1, _BK), lambda i: (0, i))],
        out_shape=[jax.ShapeDtypeStruct((_NP, _HID), f32),
                   jax.ShapeDtypeStruct((_NP, 1), f32),
                   jax.ShapeDtypeStruct((1, _NP), f32)],
        compiler_params=pltpu.CompilerParams(
            dimension_semantics=("arbitrary",)),
    )(node_feats, p['gat_W'], p['gat_b'].reshape(1, _HID),
      p['gat_v0'], p['gat_v1'])

    g = pl.pallas_call(
        _attn_body,
        grid=(_NI, _NK),
        in_specs=[pl.BlockSpec((_BM, _BK), lambda i, j: (i, j)),
                  pl.BlockSpec((_BM, 1), lambda i, j: (i, 0)),
                  pl.BlockSpec((1, _BK), lambda i, j: (0, j)),
                  pl.BlockSpec((_NP, _HID), lambda i, j: (0, 0))],
        out_specs=pl.BlockSpec((_BM, _HID), lambda i, j: (i, 0)),
        out_shape=jax.ShapeDtypeStruct((_N, _HID), f32),
        scratch_shapes=[pltpu.VMEM((_BM, _HID), f32),
                        pltpu.VMEM((_BM, 1), f32)],
        compiler_params=pltpu.CompilerParams(
            dimension_semantics=("parallel", "arbitrary")),
    )(adj_matrix, f1, f2t, h)

    sl2 = slice_label.reshape(_N, 1).astype(jnp.int32)
    full = lambda a: pl.BlockSpec(a.shape, lambda i: (0,) * a.ndim)
    smalls = [p['slice_emb'], p['gamma'], basis,
              p['enc2_W'], p['enc2_b'].reshape(1, _LAT),
              p['dec1_W'][:, :_LAT], p['dec1_W'][:, _LAT:],
              p['dec1_b'].reshape(1, _HID), p['dec2_W'],
              p['dec2_b'].reshape(1, _IN),
              p['beta_W'][:, :_LAT], p['beta_W'][:, _LAT:],
              p['beta_b'].reshape(1, _NCT),
              p['alpha_W'][:, :_LAT], p['alpha_W'][:, _LAT:],
              p['alpha_b'].reshape(1, 1)]

    dec_p, feat_p = pl.pallas_call(
        _head_body,
        grid=(_NI,),
        in_specs=[pl.BlockSpec((_BM, _HID), lambda i: (i, 0)),
                  pl.BlockSpec((_BM, _IN), lambda i: (i, 0)),
                  pl.BlockSpec((_BM, _IN), lambda i: (i, 0)),
                  pl.BlockSpec((_BM, 1), lambda i: (i, 0)),
                  pl.BlockSpec((_BM, 1), lambda i: (i, 0))]
                 + [full(a) for a in smalls],
        out_specs=[pl.BlockSpec((1, 1, 1), lambda i: (i, 0, 0)),
                   pl.BlockSpec((1, 1, 1), lambda i: (i, 0, 0))],
        out_shape=[jax.ShapeDtypeStruct((_NI, 1, 1), f32),
                   jax.ShapeDtypeStruct((_NI, 1, 1), f32)],
        compiler_params=pltpu.CompilerParams(
            dimension_semantics=("parallel",)),
    )(g, node_feats, count_matrix, library_size, sl2, *smalls)

    decon_loss = -jnp.sum(dec_p) / _N
    features_loss = jnp.sum(feat_p) / _N
    return decon_loss + _COEF_FE * features_loss


# speedup vs baseline: 1.9853x; 1.0075x over previous
import jax
import jax.numpy as jnp
from jax.experimental import pallas as pl
from jax.experimental.pallas import tpu as pltpu

_N = 10000
_BM = 1000
_NI = _N // _BM
_BK = 1024
_NK = -(-_N // _BK)
_NP = _NK * _BK
_IN = 512
_HID = 512
_LAT = 128
_NCT = 20
_NSL = 8
_SLE = 16
_COEF_FE = 0.1


def _proj_body(x_ref, W_ref, b_ref, v0_ref, v1_ref, h_ref, f1_ref, f2t_ref):
    i = pl.program_id(0)
    h = jax.lax.dot_general(x_ref[...], W_ref[...], (((1,), (1,)), ((), ())),
                            preferred_element_type=jnp.float32) + b_ref[...]
    rows = i * _BK + jax.lax.broadcasted_iota(jnp.int32, (_BK, 1), 0)
    h = jnp.where(rows < _N, h, 0.0)
    h_ref[...] = h.astype(jnp.bfloat16)
    f1_ref[...] = jnp.dot(h, v0_ref[...], preferred_element_type=jnp.float32)
    f2t_ref[...] = jax.lax.dot_general(v1_ref[...], h, (((0,), (1,)), ((), ())),
                                       preferred_element_type=jnp.float32)


def _attn_body(adj_ref, f1_ref, f2t_ref, h_ref, g_ref, acc_ref, den_ref):
    j = pl.program_id(1)

    @pl.when(j == 0)
    def _init():
        acc_ref[...] = jnp.zeros_like(acc_ref)
        den_ref[...] = jnp.zeros_like(den_ref)

    s = f1_ref[...] + f2t_ref[...]
    vals = jax.nn.sigmoid(adj_ref[...] * s) - 0.5
    cols = j * _BK + jax.lax.broadcasted_iota(jnp.int32, (1, _BK), 1)
    keep = jnp.logical_and(vals != 0.0, cols < _N)
    e = jnp.where(keep, jnp.exp(vals), 0.0)
    hj = h_ref[pl.ds(j * _BK, _BK), :]
    acc_ref[...] += jnp.dot(e.astype(jnp.bfloat16), hj,
                            preferred_element_type=jnp.float32)
    den_ref[...] += jnp.sum(e, axis=1, keepdims=True)

    @pl.when(j == _NK - 1)
    def _fin():
        g_ref[...] = acc_ref[...] / jnp.maximum(den_ref[...], 1e-30)


def _elu(x):
    return jnp.where(x > 0, x, jnp.exp(jnp.minimum(x, 0.0)) - 1.0)


def _head_body(g_ref, x_ref, cnt_ref, lib_ref, sl_ref,
               semb_ref, gam_t_ref, basis_ref,
               e2W_ref, e2b_ref,
               d1Wz_ref, d1We_ref, d1b_ref, d2W_ref, d2b_ref,
               bWz_ref, bWe_ref, bb_ref, aWz_ref, aWe_ref, ab_ref,
               dec_ref, feat_ref):
    def dg(A, B):
        return jax.lax.dot_general(A, B, (((1,), (1,)), ((), ())),
                                   preferred_element_type=jnp.float32)

    hid = _elu(g_ref[...])
    Z = dg(hid, e2W_ref[...]) + e2b_ref[...]
    oh = (sl_ref[...] == jax.lax.broadcasted_iota(jnp.int32, (_BM, _NSL), 1)
          ).astype(jnp.float32)
    emb = jnp.dot(oh, semb_ref[...], preferred_element_type=jnp.float32)
    gam = jnp.dot(oh, gam_t_ref[...], preferred_element_type=jnp.float32)
    zZ = _elu(Z)
    zE = _elu(emb)
    beta_l = dg(zZ, bWz_ref[...]) + dg(zE, bWe_ref[...]) + bb_ref[...]
    beta_l = beta_l - jnp.max(beta_l, axis=1, keepdims=True)
    ebl = jnp.exp(beta_l)
    beta = ebl / jnp.sum(ebl, axis=1, keepdims=True)
    alpha = dg(zZ, aWz_ref[...]) + dg(zE, aWe_ref[...]) + ab_ref[...]
    d1 = _elu(dg(Z, d1Wz_ref[...]) + dg(emb, d1We_ref[...]) + d1b_ref[...])
    recon = dg(d1, d2W_ref[...]) + d2b_ref[...]
    mu = jnp.dot(beta, basis_ref[...], preferred_element_type=jnp.float32)
    lib = lib_ref[...]
    log_lam = jnp.log(lib + 1e-6) + jnp.log(mu + 1e-6) + alpha + gam
    lam = lib * mu * jnp.exp(alpha) * jnp.exp(gam)
    dec_ref[...] = jnp.sum(cnt_ref[...] * log_lam - lam).reshape(1, 1, 1)
    diff = x_ref[...] - recon
    feat_ref[...] = jnp.sum(
        jnp.sqrt(jnp.sum(diff * diff, axis=1, keepdims=True) + 1e-12)
    ).reshape(1, 1, 1)


def kernel(adj_matrix, node_feats, count_matrix, library_size, slice_label,
           basis, params):
    p = params
    f32 = jnp.float32

    h, f1, f2t = pl.pallas_call(
        _proj_body,
        grid=(_NK,),
        in_specs=[pl.BlockSpec((_BK, _IN), lambda i: (i, 0)),
                  pl.BlockSpec((_HID, _IN), lambda i: (0, 0)),
                  pl.BlockSpec((1, _HID), lambda i: (0, 0)),
                  pl.BlockSpec((_HID, 1), lambda i: (0, 0)),
                  pl.BlockSpec((_HID, 1), lambda i: (0, 0))],
        out_specs=[pl.BlockSpec((_BK, _HID), lambda i: (i, 0)),
                   pl.BlockSpec((_BK, 1), lambda i: (i, 0)),
                   pl.BlockSpec((1, _BK), lambda i: (0, i))],
        out_shape=[jax.ShapeDtypeStruct((_NP, _HID), jnp.bfloat16),
                   jax.ShapeDtypeStruct((_NP, 1), f32),
                   jax.ShapeDtypeStruct((1, _NP), f32)],
        compiler_params=pltpu.CompilerParams(
            dimension_semantics=("arbitrary",)),
    )(node_feats, p['gat_W'], p['gat_b'].reshape(1, _HID),
      p['gat_v0'], p['gat_v1'])

    g = pl.pallas_call(
        _attn_body,
        grid=(_NI, _NK),
        in_specs=[pl.BlockSpec((_BM, _BK), lambda i, j: (i, j)),
                  pl.BlockSpec((_BM, 1), lambda i, j: (i, 0)),
                  pl.BlockSpec((1, _BK), lambda i, j: (0, j)),
                  pl.BlockSpec((_NP, _HID), lambda i, j: (0, 0))],
        out_specs=pl.BlockSpec((_BM, _HID), lambda i, j: (i, 0)),
        out_shape=jax.ShapeDtypeStruct((_N, _HID), f32),
        scratch_shapes=[pltpu.VMEM((_BM, _HID), f32),
                        pltpu.VMEM((_BM, 1), f32)],
        compiler_params=pltpu.CompilerParams(
            dimension_semantics=("parallel", "arbitrary")),
    )(adj_matrix, f1, f2t, h)

    sl2 = slice_label.reshape(_N, 1).astype(jnp.int32)
    full = lambda a: pl.BlockSpec(a.shape, lambda i: (0,) * a.ndim)
    smalls = [p['slice_emb'], p['gamma'], basis,
              p['enc2_W'], p['enc2_b'].reshape(1, _LAT),
              p['dec1_W'][:, :_LAT], p['dec1_W'][:, _LAT:],
              p['dec1_b'].reshape(1, _HID), p['dec2_W'],
              p['dec2_b'].reshape(1, _IN),
              p['beta_W'][:, :_LAT], p['beta_W'][:, _LAT:],
              p['beta_b'].reshape(1, _NCT),
              p['alpha_W'][:, :_LAT], p['alpha_W'][:, _LAT:],
              p['alpha_b'].reshape(1, 1)]

    dec_p, feat_p = pl.pallas_call(
        _head_body,
        grid=(_NI,),
        in_specs=[pl.BlockSpec((_BM, _HID), lambda i: (i, 0)),
                  pl.BlockSpec((_BM, _IN), lambda i: (i, 0)),
                  pl.BlockSpec((_BM, _IN), lambda i: (i, 0)),
                  pl.BlockSpec((_BM, 1), lambda i: (i, 0)),
                  pl.BlockSpec((_BM, 1), lambda i: (i, 0))]
                 + [full(a) for a in smalls],
        out_specs=[pl.BlockSpec((1, 1, 1), lambda i: (i, 0, 0)),
                   pl.BlockSpec((1, 1, 1), lambda i: (i, 0, 0))],
        out_shape=[jax.ShapeDtypeStruct((_NI, 1, 1), f32),
                   jax.ShapeDtypeStruct((_NI, 1, 1), f32)],
        compiler_params=pltpu.CompilerParams(
            dimension_semantics=("parallel",)),
    )(g, node_feats, count_matrix, library_size, sl2, *smalls)

    decon_loss = -jnp.sum(dec_p) / _N
    features_loss = jnp.sum(feat_p) / _N
    return decon_loss + _COEF_FE * features_loss
